# baseline (speedup 1.0000x reference)
import functools
import jax
import jax.numpy as jnp
from jax import lax
from jax.experimental import pallas as pl
from jax.experimental.pallas import tpu as pltpu
from jax.experimental.pallas import tpu_sc as plsc

N = 10000
E = 320000
F_IN = 128
HID = 64
C = 40
K = 10
BETA = 0.9

NC, NS, L = 2, 16, 16
NW = NC * NS
EPT = E // NW
CP = 48
CPW = 32

_SC_PARAMS = pltpu.CompilerParams(needs_layout_passes=False,
                                  use_tc_tiling_on_sc=False)
@functools.cache
def _mesh():
    return plsc.VectorSubcoreMesh(core_axis_name="c", subcore_axis_name="s",
                                  num_cores=NC, num_subcores=NS)

ROWS_BLK = 400
NBLK = N // ROWS_BLK


def _mlp_body(x_ref, w1_ref, b1_ref, w2_ref, b2_ref, z_ref, p_ref):
    x = x_ref[...]
    h = jnp.dot(x, w1_ref[...], preferred_element_type=jnp.float32,
                precision=lax.Precision.HIGHEST) + b1_ref[...]
    h = jnp.where(h >= 0, h, 0.05 * h)
    z = jnp.dot(h, w2_ref[...], preferred_element_type=jnp.float32,
                precision=lax.Precision.HIGHEST) + b2_ref[...]
    z_ref[...] = z
    m = jnp.max(z, axis=1, keepdims=True)
    e = jnp.exp(z - m)
    p_ref[...] = e / jnp.sum(e, axis=1, keepdims=True)


def _mlp(x, W1, b1, W2, b2):
    return pl.pallas_call(
        _mlp_body,
        grid=(NBLK,),
        in_specs=[
            pl.BlockSpec((ROWS_BLK, F_IN), lambda i: (i, 0)),
            pl.BlockSpec((F_IN, HID), lambda i: (0, 0)),
            pl.BlockSpec((1, HID), lambda i: (0, 0)),
            pl.BlockSpec((HID, C), lambda i: (0, 0)),
            pl.BlockSpec((1, C), lambda i: (0, 0)),
        ],
        out_specs=[
            pl.BlockSpec((ROWS_BLK, C), lambda i: (i, 0)),
            pl.BlockSpec((ROWS_BLK, C), lambda i: (i, 0)),
        ],
        out_shape=[
            jax.ShapeDtypeStruct((N, C), jnp.float32),
            jax.ShapeDtypeStruct((N, C), jnp.float32),
        ],
    )(x, W1, b1.reshape(1, HID), W2, b2.reshape(1, C))


ACH = 400
ASUB = 80
AKS = ACH // ASUB
ANCH = EPT // ACH


def _sc_attention_build():
  return functools.partial(
    pl.kernel,
    out_type=(jax.ShapeDtypeStruct((E,), jnp.float32),
              jax.ShapeDtypeStruct((NW, N), jnp.float32)),
    mesh=_mesh(),
    compiler_params=_SC_PARAMS,
    scratch_types=[
        pltpu.VMEM((EPT // ASUB, ASUB), jnp.int32),
        pltpu.VMEM((EPT // ASUB, ASUB), jnp.int32),
        pltpu.VMEM((ACH, CPW), jnp.int32),
        pltpu.VMEM((ACH, CPW), jnp.int32),
        pltpu.VMEM((ACH, CPW), jnp.int32),
        pltpu.VMEM((ACH, CPW), jnp.int32),
        pltpu.VMEM((ACH,), jnp.float32),
        pltpu.VMEM((N,), jnp.float32),
        pltpu.SemaphoreType.DMA,
        pltpu.SemaphoreType.DMA,
        pltpu.SemaphoreType.DMA,
    ],
)
def _sc_attention(p_hbm, src_hbm, dst_hbm, att_hbm, dpart_hbm,
                  src_v, dst_v, rs0, rs1, rd0, rd1, att_v, denom_v,
                  sem0, sem1, semi):
    c = lax.axis_index("c")
    s = lax.axis_index("s")
    wid = s * NC + c
    ebase = wid * EPT
    rsb = (rs0, rs1)
    rdb = (rd0, rd1)
    sems = (sem0, sem1)

    @plsc.parallel_loop(0, N // L, unroll=8)
    def zb(i):
        denom_v[pl.ds(i * L, L)] = jnp.zeros((L,), jnp.float32)

    nrows = EPT // ASUB
    cpi1 = pltpu.async_copy(src_hbm.at[pl.ds(wid * nrows, nrows)], src_v, semi)
    cpi2 = pltpu.async_copy(dst_hbm.at[pl.ds(wid * nrows, nrows)], dst_v, semi)
    cpi1.wait()
    cpi2.wait()

    def start(ch, par):
        for j in range(AKS):
            pltpu.async_copy(p_hbm.at[src_v.at[AKS * ch + j]],
                             rsb[par].at[pl.ds(j * ASUB, ASUB)], sems[par])
            pltpu.async_copy(p_hbm.at[dst_v.at[AKS * ch + j]],
                             rdb[par].at[pl.ds(j * ASUB, ASUB)], sems[par])

    def drain(par):
        for j in range(AKS):
            pltpu.make_async_copy(p_hbm.at[src_v.at[j]],
                                  rsb[par].at[pl.ds(j * ASUB, ASUB)],
                                  sems[par]).wait()
            pltpu.make_async_copy(p_hbm.at[dst_v.at[j]],
                                  rdb[par].at[pl.ds(j * ASUB, ASUB)],
                                  sems[par]).wait()

    def compute(ch, par):
        rows_s, rows_d = rsb[par], rdb[par]

        @plsc.parallel_loop(0, ACH // L, unroll=2)
        def gbody(g):
            eidx = jax.lax.broadcasted_iota(jnp.int32, (L,), 0) + g * L
            acc = jnp.zeros((L,), jnp.float32)
            for col in range(C // 2):
                cv = jnp.full((L,), col, jnp.int32)
                sv = plsc.bitcast(plsc.load_gather(rows_s, [eidx, cv]),
                                  jnp.bfloat16)
                dv = plsc.bitcast(plsc.load_gather(rows_d, [eidx, cv]),
                                  jnp.bfloat16)
                p0, p1 = plsc.unpack(sv * dv,
                                     format=plsc.PackFormat.INTERLEAVED)
                acc = acc + p0 + p1
            att_v[pl.ds(g * L, L)] = acc
            dstv = plsc.load_gather(
                dst_v, [AKS * ch + g * L // ASUB + jnp.zeros((L,), jnp.int32),
                        (g * L) % ASUB + jax.lax.broadcasted_iota(
                            jnp.int32, (L,), 0)])
            plsc.addupdate_scatter(denom_v, [dstv], acc)
        pltpu.sync_copy(att_v, att_hbm.at[pl.ds(ebase + ch * ACH, ACH)])

    start(0, 0)

    def pair(t, carry):
        chA = 2 * t
        start(chA + 1, 1)
        drain(0)
        compute(chA, 0)
        start(chA + 2, 0)
        drain(1)
        compute(chA + 1, 1)
        return carry
    lax.fori_loop(0, (ANCH - 1) // 2, pair, None)
    drain(0)
    compute(ANCH - 1, 0)

    pltpu.sync_copy(denom_v, dpart_hbm.at[wid])


DCB = 2000


def _sc_alpha_build():
  return functools.partial(
    pl.kernel,
    out_type=jax.ShapeDtypeStruct((E,), jnp.float32),
    mesh=_mesh(),
    compiler_params=_SC_PARAMS,
    scratch_types=[
        pltpu.VMEM((NW, DCB), jnp.float32),
        pltpu.VMEM((N,), jnp.float32),
        pltpu.VMEM((ACH,), jnp.int32),
        pltpu.VMEM((ACH,), jnp.float32),
        pltpu.VMEM((ACH,), jnp.float32),
        pltpu.SemaphoreType.DMA,
    ],
)
def _sc_alpha(dpart_hbm, dst_hbm, att_hbm, alpha_hbm,
              dchunk_v, denom_v, dst_v, att_v, al_v, sem):
    c = lax.axis_index("c")
    s = lax.axis_index("s")
    wid = s * NC + c
    ebase = wid * EPT

    def comb(cb, carry):
        pltpu.sync_copy(dpart_hbm.at[:, pl.ds(cb * DCB, DCB)], dchunk_v)

        @plsc.parallel_loop(0, DCB // L, unroll=2)
        def cbody(v):
            acc = jnp.zeros((L,), jnp.float32)
            for w in range(NW):
                acc = acc + dchunk_v[w, pl.ds(v * L, L)]
            denom_v[pl.ds(cb * DCB + v * L, L)] = acc
        return carry
    lax.fori_loop(0, N // DCB, comb, None)

    def chunk(ch, carry):
        base = ebase + ch * ACH
        cp1 = pltpu.async_copy(dst_hbm.at[pl.ds(base, ACH)], dst_v, sem)
        cp2 = pltpu.async_copy(att_hbm.at[pl.ds(base, ACH)], att_v, sem)
        cp1.wait()
        cp2.wait()

        @plsc.parallel_loop(0, ACH // L, unroll=4)
        def gbody(g):
            dstv = dst_v[pl.ds(g * L, L)]
            attv = att_v[pl.ds(g * L, L)]
            d = plsc.load_gather(denom_v, [dstv])
            al_v[pl.ds(g * L, L)] = attv / (d + 1e-16)
        pltpu.sync_copy(al_v, alpha_hbm.at[pl.ds(base, ACH)])
        return carry
    lax.fori_loop(0, ANCH, chunk, None)


FPT = 2
NACT = C // FPT
PCH = 10000
PNCH = E // PCH


def _sc_diffuse_build():
  return functools.partial(
    pl.kernel,
    out_type=jax.ShapeDtypeStruct((C, N), jnp.float32),
    mesh=_mesh(),
    compiler_params=_SC_PARAMS,
    scratch_types=[
        pltpu.VMEM((FPT, N), jnp.float32),
        pltpu.VMEM((N,), jnp.int32),
        pltpu.VMEM((FPT, N), jnp.float32),
        pltpu.VMEM((PCH,), jnp.int32),
        pltpu.VMEM((PCH,), jnp.int32),
        pltpu.VMEM((PCH,), jnp.float32),
        pltpu.VMEM((PCH,), jnp.float32),
        pltpu.SemaphoreType.DMA,
        pltpu.SemaphoreType.DMA,
    ],
)
def _sc_diffuse(zT_hbm, sd_hbm, alpha_hbm, outT_hbm,
                z_v, out_p, agg_v, sd0, sd1, al0, al1,
                sem0, sem1):
    c = lax.axis_index("c")
    s = lax.axis_index("s")
    wid = s * NC + c
    sdb = (sd0, sd1)
    alb = (al0, al1)
    sems = (sem0, sem1)

    @pl.when(wid < NACT)
    def _():
        pltpu.sync_copy(zT_hbm.at[pl.ds(FPT * wid, FPT)], z_v)

        @plsc.parallel_loop(0, N // L, unroll=8)
        def icopy(i):
            pk = plsc.pack(z_v[0, pl.ds(i * L, L)], z_v[1, pl.ds(i * L, L)],
                           format=plsc.PackFormat.INTERLEAVED)
            out_p[pl.ds(i * L, L)] = plsc.bitcast(pk, jnp.int32)

        def start(ch, par):
            base = ch * PCH
            return [
                pltpu.async_copy(sd_hbm.at[pl.ds(base, PCH)], sdb[par],
                                 sems[par]),
                pltpu.async_copy(alpha_hbm.at[pl.ds(base, PCH)], alb[par],
                                 sems[par]),
            ]

        def drain(par):
            pltpu.make_async_copy(sd_hbm.at[pl.ds(0, PCH)], sdb[par],
                                  sems[par]).wait()
            pltpu.make_async_copy(alpha_hbm.at[pl.ds(0, PCH)], alb[par],
                                  sems[par]).wait()

        def compute(par):
            sref, aref = sdb[par], alb[par]

            @plsc.parallel_loop(0, PCH // L, unroll=8)
            def gbody(g):
                sdv = sref[pl.ds(g * L, L)]
                srcv = lax.shift_right_logical(sdv, 14)
                dstv = lax.bitwise_and(sdv, 16383)
                av = aref[pl.ds(g * L, L)]
                pair = plsc.load_gather(out_p, [srcv])
                vals = plsc.bitcast(pair, jnp.bfloat16)
                avd = plsc.pack(av, av, format=plsc.PackFormat.INTERLEAVED)
                m0, m1 = plsc.unpack(vals * avd,
                                     format=plsc.PackFormat.INTERLEAVED)
                plsc.addupdate_scatter(
                    agg_v, [jnp.zeros((L,), jnp.int32), dstv], m0)
                plsc.addupdate_scatter(
                    agg_v, [jnp.full((L,), 1, jnp.int32), dstv], m1)

        def step(it, carry):
            @plsc.parallel_loop(0, N // L, unroll=8)
            def zb(i):
                z16 = jnp.zeros((L,), jnp.float32)
                for r in range(FPT):
                    agg_v[r, pl.ds(i * L, L)] = z16

            start(0, 0)

            def pair(t, cc):
                chA = 2 * t
                start(chA + 1, 1)
                drain(0)
                compute(0)
                start(lax.rem(chA + 2, PNCH), 0)
                drain(1)
                compute(1)
                return cc
            lax.fori_loop(0, PNCH // 2, pair, None)
            drain(0)

            @plsc.parallel_loop(0, N // L, unroll=8)
            def ub(i):
                o0 = (BETA * agg_v[0, pl.ds(i * L, L)] +
                      (1.0 - BETA) * z_v[0, pl.ds(i * L, L)])
                o1 = (BETA * agg_v[1, pl.ds(i * L, L)] +
                      (1.0 - BETA) * z_v[1, pl.ds(i * L, L)])
                pk = plsc.pack(o0, o1, format=plsc.PackFormat.INTERLEAVED)
                out_p[pl.ds(i * L, L)] = plsc.bitcast(pk, jnp.int32)
            return carry

        lax.fori_loop(0, K, step, None)

        @plsc.parallel_loop(0, N // L, unroll=8)
        def fin(i):
            for r in range(FPT):
                agg_v[r, pl.ds(i * L, L)] = (
                    BETA * agg_v[r, pl.ds(i * L, L)] +
                    (1.0 - BETA) * z_v[r, pl.ds(i * L, L)])
        pltpu.sync_copy(agg_v, outT_hbm.at[pl.ds(FPT * wid, FPT)])


def _post_body(o_ref, logp_ref, ent_ref):
    i = pl.program_id(0)
    o = o_ref[...]
    m = jnp.max(o, axis=1, keepdims=True)
    ex = jnp.exp(o - m)
    se = jnp.sum(ex, axis=1, keepdims=True)
    logp = o - m - jnp.log(se)
    logp_ref[...] = logp
    q = ex / se
    ent_blk = -jnp.sum(q * jnp.log(q + 1e-16))

    @pl.when(i == 0)
    def _():
        ent_ref[...] = jnp.zeros((1, 1), jnp.float32)

    ent_ref[...] += jnp.full((1, 1), ent_blk, jnp.float32)

    @pl.when(i == NBLK - 1)
    def _():
        ent_ref[...] = ent_ref[...] / N


def _post(out_mat):
    return pl.pallas_call(
        _post_body,
        grid=(NBLK,),
        in_specs=[pl.BlockSpec((ROWS_BLK, C), lambda i: (i, 0))],
        out_specs=[
            pl.BlockSpec((ROWS_BLK, C), lambda i: (i, 0)),
            pl.BlockSpec((1, 1), lambda i: (0, 0)),
        ],
        out_shape=[
            jax.ShapeDtypeStruct((N, C), jnp.float32),
            jax.ShapeDtypeStruct((1, 1), jnp.float32),
        ],
    )(out_mat)


def kernel(x, edge_index, train_mask, W1, b1, W2, b2, is_debug):
    z, p = _mlp(x, W1, b1, W2, b2)
    src = edge_index[0]
    dst = edge_index[1]
    pb = jnp.pad(p, ((0, 0), (0, 2 * CPW - C))).astype(jnp.bfloat16)
    pu = jax.lax.bitcast_convert_type(pb, jnp.uint16).astype(jnp.uint32)
    p_pack = (pu[:, 0::2] | (pu[:, 1::2] << 16)).astype(jnp.int32)
    att, dpart = _sc_attention_build()(_sc_attention)(
        p_pack, src.reshape(E // ASUB, ASUB), dst.reshape(E // ASUB, ASUB))
    alpha = _sc_alpha_build()(_sc_alpha)(dpart, dst, att)
    zT = z.T
    sd = jnp.left_shift(src, 14) | dst
    outT = _sc_diffuse_build()(_sc_diffuse)(zT, sd, alpha)
    logp, ent11 = _post(outT.T)
    return (logp, ent11[0, 0], att)


# speedup vs baseline: 1.0031x; 1.0031x over previous
import functools
import jax
import jax.numpy as jnp
from jax import lax
from jax.experimental import pallas as pl
from jax.experimental.pallas import tpu as pltpu
from jax.experimental.pallas import tpu_sc as plsc

N = 10000
E = 320000
F_IN = 128
HID = 64
C = 40
K = 10
BETA = 0.9

NC, NS, L = 2, 16, 16
NW = NC * NS
EPT = E // NW
CP = 48
CPW = 32

_SC_PARAMS = pltpu.CompilerParams(needs_layout_passes=False,
                                  use_tc_tiling_on_sc=False)
@functools.cache
def _mesh():
    return plsc.VectorSubcoreMesh(core_axis_name="c", subcore_axis_name="s",
                                  num_cores=NC, num_subcores=NS)

ROWS_BLK = 400
NBLK = N // ROWS_BLK


def _mlp_body(x_ref, w1_ref, b1_ref, w2_ref, b2_ref, zt_ref, pp_ref):
    x = x_ref[...]
    h = jnp.dot(x, w1_ref[...], preferred_element_type=jnp.float32,
                precision=lax.Precision.HIGHEST) + b1_ref[...]
    h = jnp.where(h >= 0, h, 0.05 * h)
    z = jnp.dot(h, w2_ref[...], preferred_element_type=jnp.float32,
                precision=lax.Precision.HIGHEST) + b2_ref[...]
    zt_ref[...] = z.T
    m = jnp.max(z, axis=1, keepdims=True)
    e = jnp.exp(z - m)
    p = e / jnp.sum(e, axis=1, keepdims=True)
    pb = jnp.concatenate(
        [p, jnp.zeros((N, 2 * CPW - C), jnp.float32)], axis=1)
    pb = pb.astype(jnp.bfloat16)
    u = lax.bitcast_convert_type(pb, jnp.uint16).astype(jnp.uint32)
    pp_ref[...] = (u[:, :CPW] | (u[:, CPW:] << 16)).astype(jnp.int32)


def _mlp(x, W1, b1, W2, b2):
    return pl.pallas_call(
        _mlp_body,
        out_shape=[
            jax.ShapeDtypeStruct((C, N), jnp.float32),
            jax.ShapeDtypeStruct((N, CPW), jnp.int32),
        ],
    )(x, W1, b1.reshape(1, HID), W2, b2.reshape(1, C))


ACH = 400
ASUB = 80
AKS = ACH // ASUB
ANCH = EPT // ACH


def _sc_attention_build():
  return functools.partial(
    pl.kernel,
    out_type=(jax.ShapeDtypeStruct((E,), jnp.float32),
              jax.ShapeDtypeStruct((NW, N), jnp.float32)),
    mesh=_mesh(),
    compiler_params=_SC_PARAMS,
    scratch_types=[
        pltpu.VMEM((EPT // ASUB, ASUB), jnp.int32),
        pltpu.VMEM((EPT // ASUB, ASUB), jnp.int32),
        pltpu.VMEM((ACH, CPW), jnp.int32),
        pltpu.VMEM((ACH, CPW), jnp.int32),
        pltpu.VMEM((ACH, CPW), jnp.int32),
        pltpu.VMEM((ACH, CPW), jnp.int32),
        pltpu.VMEM((ACH,), jnp.float32),
        pltpu.VMEM((N,), jnp.float32),
        pltpu.SemaphoreType.DMA,
        pltpu.SemaphoreType.DMA,
        pltpu.SemaphoreType.DMA,
    ],
)
def _sc_attention(p_hbm, src_hbm, dst_hbm, att_hbm, dpart_hbm,
                  src_v, dst_v, rs0, rs1, rd0, rd1, att_v, denom_v,
                  sem0, sem1, semi):
    c = lax.axis_index("c")
    s = lax.axis_index("s")
    wid = s * NC + c
    ebase = wid * EPT
    rsb = (rs0, rs1)
    rdb = (rd0, rd1)
    sems = (sem0, sem1)

    @plsc.parallel_loop(0, N // L, unroll=8)
    def zb(i):
        denom_v[pl.ds(i * L, L)] = jnp.zeros((L,), jnp.float32)

    nrows = EPT // ASUB
    cpi1 = pltpu.async_copy(src_hbm.at[pl.ds(wid * nrows, nrows)], src_v, semi)
    cpi2 = pltpu.async_copy(dst_hbm.at[pl.ds(wid * nrows, nrows)], dst_v, semi)
    cpi1.wait()
    cpi2.wait()

    def start(ch, par):
        for j in range(AKS):
            pltpu.async_copy(p_hbm.at[src_v.at[AKS * ch + j]],
                             rsb[par].at[pl.ds(j * ASUB, ASUB)], sems[par])
            pltpu.async_copy(p_hbm.at[dst_v.at[AKS * ch + j]],
                             rdb[par].at[pl.ds(j * ASUB, ASUB)], sems[par])

    def drain(par):
        for j in range(AKS):
            pltpu.make_async_copy(p_hbm.at[src_v.at[j]],
                                  rsb[par].at[pl.ds(j * ASUB, ASUB)],
                                  sems[par]).wait()
            pltpu.make_async_copy(p_hbm.at[dst_v.at[j]],
                                  rdb[par].at[pl.ds(j * ASUB, ASUB)],
                                  sems[par]).wait()

    def compute(ch, par):
        rows_s, rows_d = rsb[par], rdb[par]

        @plsc.parallel_loop(0, ACH // L, unroll=2)
        def gbody(g):
            eidx = jax.lax.broadcasted_iota(jnp.int32, (L,), 0) + g * L
            acc = jnp.zeros((L,), jnp.float32)
            for col in range(CPW):
                cv = jnp.full((L,), col, jnp.int32)
                sv = plsc.bitcast(plsc.load_gather(rows_s, [eidx, cv]),
                                  jnp.bfloat16)
                dv = plsc.bitcast(plsc.load_gather(rows_d, [eidx, cv]),
                                  jnp.bfloat16)
                p0, p1 = plsc.unpack(sv * dv,
                                     format=plsc.PackFormat.INTERLEAVED)
                acc = acc + p0 + p1
            att_v[pl.ds(g * L, L)] = acc
            dstv = plsc.load_gather(
                dst_v, [AKS * ch + g * L // ASUB + jnp.zeros((L,), jnp.int32),
                        (g * L) % ASUB + jax.lax.broadcasted_iota(
                            jnp.int32, (L,), 0)])
            plsc.addupdate_scatter(denom_v, [dstv], acc)
        pltpu.sync_copy(att_v, att_hbm.at[pl.ds(ebase + ch * ACH, ACH)])

    start(0, 0)

    def pair(t, carry):
        chA = 2 * t
        start(chA + 1, 1)
        drain(0)
        compute(chA, 0)
        start(chA + 2, 0)
        drain(1)
        compute(chA + 1, 1)
        return carry
    lax.fori_loop(0, (ANCH - 1) // 2, pair, None)
    drain(0)
    compute(ANCH - 1, 0)

    pltpu.sync_copy(denom_v, dpart_hbm.at[wid])


DCB = 2000


def _sc_alpha_build():
  return functools.partial(
    pl.kernel,
    out_type=jax.ShapeDtypeStruct((E,), jnp.float32),
    mesh=_mesh(),
    compiler_params=_SC_PARAMS,
    scratch_types=[
        pltpu.VMEM((NW, DCB), jnp.float32),
        pltpu.VMEM((N,), jnp.float32),
        pltpu.VMEM((ACH,), jnp.int32),
        pltpu.VMEM((ACH,), jnp.float32),
        pltpu.VMEM((ACH,), jnp.float32),
        pltpu.SemaphoreType.DMA,
    ],
)
def _sc_alpha(dpart_hbm, dst_hbm, att_hbm, alpha_hbm,
              dchunk_v, denom_v, dst_v, att_v, al_v, sem):
    c = lax.axis_index("c")
    s = lax.axis_index("s")
    wid = s * NC + c
    ebase = wid * EPT

    def comb(cb, carry):
        pltpu.sync_copy(dpart_hbm.at[:, pl.ds(cb * DCB, DCB)], dchunk_v)

        @plsc.parallel_loop(0, DCB // L, unroll=2)
        def cbody(v):
            acc = jnp.zeros((L,), jnp.float32)
            for w in range(NW):
                acc = acc + dchunk_v[w, pl.ds(v * L, L)]
            denom_v[pl.ds(cb * DCB + v * L, L)] = acc
        return carry
    lax.fori_loop(0, N // DCB, comb, None)

    def chunk(ch, carry):
        base = ebase + ch * ACH
        cp1 = pltpu.async_copy(dst_hbm.at[pl.ds(base, ACH)], dst_v, sem)
        cp2 = pltpu.async_copy(att_hbm.at[pl.ds(base, ACH)], att_v, sem)
        cp1.wait()
        cp2.wait()

        @plsc.parallel_loop(0, ACH // L, unroll=4)
        def gbody(g):
            dstv = dst_v[pl.ds(g * L, L)]
            attv = att_v[pl.ds(g * L, L)]
            d = plsc.load_gather(denom_v, [dstv])
            al_v[pl.ds(g * L, L)] = attv / (d + 1e-16)
        pltpu.sync_copy(al_v, alpha_hbm.at[pl.ds(base, ACH)])
        return carry
    lax.fori_loop(0, ANCH, chunk, None)


FPT = 2
NACT = C // FPT
PCH = 10000
PNCH = E // PCH


def _sc_diffuse_build():
  return functools.partial(
    pl.kernel,
    out_type=jax.ShapeDtypeStruct((C, N), jnp.float32),
    mesh=_mesh(),
    compiler_params=_SC_PARAMS,
    scratch_types=[
        pltpu.VMEM((FPT, N), jnp.float32),
        pltpu.VMEM((N,), jnp.int32),
        pltpu.VMEM((FPT, N), jnp.float32),
        pltpu.VMEM((PCH,), jnp.int32),
        pltpu.VMEM((PCH,), jnp.int32),
        pltpu.VMEM((PCH,), jnp.float32),
        pltpu.VMEM((PCH,), jnp.float32),
        pltpu.SemaphoreType.DMA,
        pltpu.SemaphoreType.DMA,
    ],
)
def _sc_diffuse(zT_hbm, sd_hbm, alpha_hbm, outT_hbm,
                z_v, out_p, agg_v, sd0, sd1, al0, al1,
                sem0, sem1):
    c = lax.axis_index("c")
    s = lax.axis_index("s")
    wid = s * NC + c
    sdb = (sd0, sd1)
    alb = (al0, al1)
    sems = (sem0, sem1)

    @pl.when(wid < NACT)
    def _():
        pltpu.sync_copy(zT_hbm.at[pl.ds(FPT * wid, FPT)], z_v)

        @plsc.parallel_loop(0, N // L, unroll=8)
        def icopy(i):
            pk = plsc.pack(z_v[0, pl.ds(i * L, L)], z_v[1, pl.ds(i * L, L)],
                           format=plsc.PackFormat.INTERLEAVED)
            out_p[pl.ds(i * L, L)] = plsc.bitcast(pk, jnp.int32)

        def start(ch, par):
            base = ch * PCH
            return [
                pltpu.async_copy(sd_hbm.at[pl.ds(base, PCH)], sdb[par],
                                 sems[par]),
                pltpu.async_copy(alpha_hbm.at[pl.ds(base, PCH)], alb[par],
                                 sems[par]),
            ]

        def drain(par):
            pltpu.make_async_copy(sd_hbm.at[pl.ds(0, PCH)], sdb[par],
                                  sems[par]).wait()
            pltpu.make_async_copy(alpha_hbm.at[pl.ds(0, PCH)], alb[par],
                                  sems[par]).wait()

        def compute(par):
            sref, aref = sdb[par], alb[par]

            @plsc.parallel_loop(0, PCH // L, unroll=8)
            def gbody(g):
                sdv = sref[pl.ds(g * L, L)]
                srcv = lax.shift_right_logical(sdv, 14)
                dstv = lax.bitwise_and(sdv, 16383)
                av = aref[pl.ds(g * L, L)]
                pair = plsc.load_gather(out_p, [srcv])
                vals = plsc.bitcast(pair, jnp.bfloat16)
                avd = plsc.pack(av, av, format=plsc.PackFormat.INTERLEAVED)
                m0, m1 = plsc.unpack(vals * avd,
                                     format=plsc.PackFormat.INTERLEAVED)
                plsc.addupdate_scatter(
                    agg_v, [jnp.zeros((L,), jnp.int32), dstv], m0)
                plsc.addupdate_scatter(
                    agg_v, [jnp.full((L,), 1, jnp.int32), dstv], m1)

        def step(it, carry):
            @plsc.parallel_loop(0, N // L, unroll=8)
            def zb(i):
                z16 = jnp.zeros((L,), jnp.float32)
                for r in range(FPT):
                    agg_v[r, pl.ds(i * L, L)] = z16

            start(0, 0)

            def pair(t, cc):
                chA = 2 * t
                start(chA + 1, 1)
                drain(0)
                compute(0)
                start(lax.rem(chA + 2, PNCH), 0)
                drain(1)
                compute(1)
                return cc
            lax.fori_loop(0, PNCH // 2, pair, None)
            drain(0)

            @plsc.parallel_loop(0, N // L, unroll=8)
            def ub(i):
                o0 = (BETA * agg_v[0, pl.ds(i * L, L)] +
                      (1.0 - BETA) * z_v[0, pl.ds(i * L, L)])
                o1 = (BETA * agg_v[1, pl.ds(i * L, L)] +
                      (1.0 - BETA) * z_v[1, pl.ds(i * L, L)])
                pk = plsc.pack(o0, o1, format=plsc.PackFormat.INTERLEAVED)
                out_p[pl.ds(i * L, L)] = plsc.bitcast(pk, jnp.int32)
            return carry

        lax.fori_loop(0, K, step, None)

        @plsc.parallel_loop(0, N // L, unroll=8)
        def fin(i):
            for r in range(FPT):
                agg_v[r, pl.ds(i * L, L)] = (
                    BETA * agg_v[r, pl.ds(i * L, L)] +
                    (1.0 - BETA) * z_v[r, pl.ds(i * L, L)])
        pltpu.sync_copy(agg_v, outT_hbm.at[pl.ds(FPT * wid, FPT)])


def _post_body(ot_ref, logp_ref, ent_ref):
    o = ot_ref[...].T
    m = jnp.max(o, axis=1, keepdims=True)
    ex = jnp.exp(o - m)
    se = jnp.sum(ex, axis=1, keepdims=True)
    logp = o - m - jnp.log(se)
    logp_ref[...] = logp
    q = ex / se
    ent = -jnp.sum(q * jnp.log(q + 1e-16)) / N
    ent_ref[...] = jnp.full((1, 1), ent, jnp.float32)


def _post(outT):
    return pl.pallas_call(
        _post_body,
        out_shape=[
            jax.ShapeDtypeStruct((N, C), jnp.float32),
            jax.ShapeDtypeStruct((1, 1), jnp.float32),
        ],
    )(outT)


def kernel(x, edge_index, train_mask, W1, b1, W2, b2, is_debug):
    zT, p_pack = _mlp(x, W1, b1, W2, b2)
    src = edge_index[0]
    dst = edge_index[1]
    att, dpart = _sc_attention_build()(_sc_attention)(
        p_pack, src.reshape(E // ASUB, ASUB), dst.reshape(E // ASUB, ASUB))
    alpha = _sc_alpha_build()(_sc_alpha)(dpart, dst, att)
    sd = jnp.left_shift(src, 14) | dst
    outT = _sc_diffuse_build()(_sc_diffuse)(zT, sd, alpha)
    logp, ent11 = _post(outT)
    return (logp, ent11[0, 0], att)


# speedup vs baseline: 1.0800x; 1.0767x over previous
import functools
import jax
import jax.numpy as jnp
from jax import lax
from jax.experimental import pallas as pl
from jax.experimental.pallas import tpu as pltpu
from jax.experimental.pallas import tpu_sc as plsc

N = 10000
E = 320000
F_IN = 128
HID = 64
C = 40
K = 10
BETA = 0.9

NC, NS, L = 2, 16, 16
NW = NC * NS
EPT = E // NW
CP = 48
CPW = 32

_SC_PARAMS = pltpu.CompilerParams(needs_layout_passes=False,
                                  use_tc_tiling_on_sc=False)
@functools.cache
def _mesh():
    return plsc.VectorSubcoreMesh(core_axis_name="c", subcore_axis_name="s",
                                  num_cores=NC, num_subcores=NS)

ROWS_BLK = 400
NBLK = N // ROWS_BLK


def _mlp_body(x_ref, w1_ref, b1_ref, w2_ref, b2_ref, zt_ref, pp_ref):
    x = x_ref[...]
    h = jnp.dot(x, w1_ref[...], preferred_element_type=jnp.float32,
                precision=lax.Precision.HIGHEST) + b1_ref[...]
    h = jnp.where(h >= 0, h, 0.05 * h)
    z = jnp.dot(h, w2_ref[...], preferred_element_type=jnp.float32,
                precision=lax.Precision.HIGHEST) + b2_ref[...]
    zt_ref[...] = z.T
    m = jnp.max(z, axis=1, keepdims=True)
    e = jnp.exp(z - m)
    p = e / jnp.sum(e, axis=1, keepdims=True)
    u = lax.bitcast_convert_type(p.astype(jnp.bfloat16),
                                 jnp.uint16).astype(jnp.uint32)
    words = u[:, :C // 2] | (u[:, C // 2:] << 16)
    pp_ref[...] = jnp.concatenate(
        [words, jnp.zeros((N, CPW - C // 2), jnp.uint32)],
        axis=1).astype(jnp.int32)


def _mlp(x, W1, b1, W2, b2):
    return pl.pallas_call(
        _mlp_body,
        out_shape=[
            jax.ShapeDtypeStruct((C, N), jnp.float32),
            jax.ShapeDtypeStruct((N, CPW), jnp.int32),
        ],
    )(x, W1, b1.reshape(1, HID), W2, b2.reshape(1, C))


ACH = 400
ASUB = 80
AKS = ACH // ASUB
ANCH = EPT // ACH


def _sc_attention_build():
  return functools.partial(
    pl.kernel,
    out_type=(jax.ShapeDtypeStruct((E,), jnp.float32),
              jax.ShapeDtypeStruct((NW, N), jnp.float32)),
    mesh=_mesh(),
    compiler_params=_SC_PARAMS,
    scratch_types=[
        pltpu.VMEM((EPT // ASUB, ASUB), jnp.int32),
        pltpu.VMEM((EPT // ASUB, ASUB), jnp.int32),
        pltpu.VMEM((ACH, CPW), jnp.int32),
        pltpu.VMEM((ACH, CPW), jnp.int32),
        pltpu.VMEM((ACH, CPW), jnp.int32),
        pltpu.VMEM((ACH, CPW), jnp.int32),
        pltpu.VMEM((ACH,), jnp.float32),
        pltpu.VMEM((N,), jnp.float32),
        pltpu.SemaphoreType.DMA,
        pltpu.SemaphoreType.DMA,
        pltpu.SemaphoreType.DMA,
    ],
)
def _sc_attention(p_hbm, src_hbm, dst_hbm, att_hbm, dpart_hbm,
                  src_v, dst_v, rs0, rs1, rd0, rd1, att_v, denom_v,
                  sem0, sem1, semi):
    c = lax.axis_index("c")
    s = lax.axis_index("s")
    wid = s * NC + c
    ebase = wid * EPT
    rsb = (rs0, rs1)
    rdb = (rd0, rd1)
    sems = (sem0, sem1)

    @plsc.parallel_loop(0, N // L, unroll=8)
    def zb(i):
        denom_v[pl.ds(i * L, L)] = jnp.zeros((L,), jnp.float32)

    nrows = EPT // ASUB
    cpi1 = pltpu.async_copy(src_hbm.at[pl.ds(wid * nrows, nrows)], src_v, semi)
    cpi2 = pltpu.async_copy(dst_hbm.at[pl.ds(wid * nrows, nrows)], dst_v, semi)
    cpi1.wait()
    cpi2.wait()

    def start(ch, par):
        for j in range(AKS):
            pltpu.async_copy(p_hbm.at[src_v.at[AKS * ch + j]],
                             rsb[par].at[pl.ds(j * ASUB, ASUB)], sems[par])
            pltpu.async_copy(p_hbm.at[dst_v.at[AKS * ch + j]],
                             rdb[par].at[pl.ds(j * ASUB, ASUB)], sems[par])

    def drain(par):
        for j in range(AKS):
            pltpu.make_async_copy(p_hbm.at[src_v.at[j]],
                                  rsb[par].at[pl.ds(j * ASUB, ASUB)],
                                  sems[par]).wait()
            pltpu.make_async_copy(p_hbm.at[dst_v.at[j]],
                                  rdb[par].at[pl.ds(j * ASUB, ASUB)],
                                  sems[par]).wait()

    def compute(ch, par):
        rows_s, rows_d = rsb[par], rdb[par]

        @plsc.parallel_loop(0, ACH // L, unroll=2)
        def gbody(g):
            eidx = jax.lax.broadcasted_iota(jnp.int32, (L,), 0) + g * L
            acc = jnp.zeros((L,), jnp.float32)
            for col in range(C // 2):
                cv = jnp.full((L,), col, jnp.int32)
                sv = plsc.bitcast(plsc.load_gather(rows_s, [eidx, cv]),
                                  jnp.bfloat16)
                dv = plsc.bitcast(plsc.load_gather(rows_d, [eidx, cv]),
                                  jnp.bfloat16)
                p0, p1 = plsc.unpack(sv * dv,
                                     format=plsc.PackFormat.INTERLEAVED)
                acc = acc + p0 + p1
            att_v[pl.ds(g * L, L)] = acc
            dstv = plsc.load_gather(
                dst_v, [AKS * ch + g * L // ASUB + jnp.zeros((L,), jnp.int32),
                        (g * L) % ASUB + jax.lax.broadcasted_iota(
                            jnp.int32, (L,), 0)])
            plsc.addupdate_scatter(denom_v, [dstv], acc)
        pltpu.sync_copy(att_v, att_hbm.at[pl.ds(ebase + ch * ACH, ACH)])

    start(0, 0)

    def pair(t, carry):
        chA = 2 * t
        start(chA + 1, 1)
        drain(0)
        compute(chA, 0)
        start(chA + 2, 0)
        drain(1)
        compute(chA + 1, 1)
        return carry
    lax.fori_loop(0, (ANCH - 1) // 2, pair, None)
    drain(0)
    compute(ANCH - 1, 0)

    pltpu.sync_copy(denom_v, dpart_hbm.at[wid])


DCB = 2000


def _sc_alpha_build():
  return functools.partial(
    pl.kernel,
    out_type=jax.ShapeDtypeStruct((E,), jnp.float32),
    mesh=_mesh(),
    compiler_params=_SC_PARAMS,
    scratch_types=[
        pltpu.VMEM((NW, DCB), jnp.float32),
        pltpu.VMEM((N,), jnp.float32),
        pltpu.VMEM((ACH,), jnp.int32),
        pltpu.VMEM((ACH,), jnp.float32),
        pltpu.VMEM((ACH,), jnp.float32),
        pltpu.SemaphoreType.DMA,
    ],
)
def _sc_alpha(dpart_hbm, dst_hbm, att_hbm, alpha_hbm,
              dchunk_v, denom_v, dst_v, att_v, al_v, sem):
    c = lax.axis_index("c")
    s = lax.axis_index("s")
    wid = s * NC + c
    ebase = wid * EPT

    def comb(cb, carry):
        pltpu.sync_copy(dpart_hbm.at[:, pl.ds(cb * DCB, DCB)], dchunk_v)

        @plsc.parallel_loop(0, DCB // L, unroll=2)
        def cbody(v):
            acc = jnp.zeros((L,), jnp.float32)
            for w in range(NW):
                acc = acc + dchunk_v[w, pl.ds(v * L, L)]
            denom_v[pl.ds(cb * DCB + v * L, L)] = acc
        return carry
    lax.fori_loop(0, N // DCB, comb, None)

    def chunk(ch, carry):
        base = ebase + ch * ACH
        cp1 = pltpu.async_copy(dst_hbm.at[pl.ds(base, ACH)], dst_v, sem)
        cp2 = pltpu.async_copy(att_hbm.at[pl.ds(base, ACH)], att_v, sem)
        cp1.wait()
        cp2.wait()

        @plsc.parallel_loop(0, ACH // L, unroll=4)
        def gbody(g):
            dstv = dst_v[pl.ds(g * L, L)]
            attv = att_v[pl.ds(g * L, L)]
            d = plsc.load_gather(denom_v, [dstv])
            al_v[pl.ds(g * L, L)] = attv / (d + 1e-16)
        pltpu.sync_copy(al_v, alpha_hbm.at[pl.ds(base, ACH)])
        return carry
    lax.fori_loop(0, ANCH, chunk, None)


FPT = 2
NACT = C // FPT
PCH = 10000
PNCH = E // PCH


def _sc_diffuse_build():
  return functools.partial(
    pl.kernel,
    out_type=jax.ShapeDtypeStruct((C, N), jnp.float32),
    mesh=_mesh(),
    compiler_params=_SC_PARAMS,
    scratch_types=[
        pltpu.VMEM((FPT, N), jnp.float32),
        pltpu.VMEM((N,), jnp.int32),
        pltpu.VMEM((FPT, N), jnp.float32),
        pltpu.VMEM((PCH,), jnp.int32),
        pltpu.VMEM((PCH,), jnp.int32),
        pltpu.VMEM((PCH,), jnp.float32),
        pltpu.VMEM((PCH,), jnp.float32),
        pltpu.SemaphoreType.DMA,
        pltpu.SemaphoreType.DMA,
    ],
)
def _sc_diffuse(zT_hbm, sd_hbm, alpha_hbm, outT_hbm,
                z_v, out_p, agg_v, sd0, sd1, al0, al1,
                sem0, sem1):
    c = lax.axis_index("c")
    s = lax.axis_index("s")
    wid = s * NC + c
    sdb = (sd0, sd1)
    alb = (al0, al1)
    sems = (sem0, sem1)

    @pl.when(wid < NACT)
    def _():
        pltpu.sync_copy(zT_hbm.at[pl.ds(FPT * wid, FPT)], z_v)

        @plsc.parallel_loop(0, N // L, unroll=8)
        def icopy(i):
            pk = plsc.pack(z_v[0, pl.ds(i * L, L)], z_v[1, pl.ds(i * L, L)],
                           format=plsc.PackFormat.INTERLEAVED)
            out_p[pl.ds(i * L, L)] = plsc.bitcast(pk, jnp.int32)

        def start(ch, par):
            base = ch * PCH
            return [
                pltpu.async_copy(sd_hbm.at[pl.ds(base, PCH)], sdb[par],
                                 sems[par]),
                pltpu.async_copy(alpha_hbm.at[pl.ds(base, PCH)], alb[par],
                                 sems[par]),
            ]

        def drain(par):
            pltpu.make_async_copy(sd_hbm.at[pl.ds(0, PCH)], sdb[par],
                                  sems[par]).wait()
            pltpu.make_async_copy(alpha_hbm.at[pl.ds(0, PCH)], alb[par],
                                  sems[par]).wait()

        def compute(par):
            sref, aref = sdb[par], alb[par]

            @plsc.parallel_loop(0, PCH // L, unroll=8)
            def gbody(g):
                sdv = sref[pl.ds(g * L, L)]
                srcv = lax.shift_right_logical(sdv, 14)
                dstv = lax.bitwise_and(sdv, 16383)
                av = aref[pl.ds(g * L, L)]
                pair = plsc.load_gather(out_p, [srcv])
                vals = plsc.bitcast(pair, jnp.bfloat16)
                avd = plsc.pack(av, av, format=plsc.PackFormat.INTERLEAVED)
                m0, m1 = plsc.unpack(vals * avd,
                                     format=plsc.PackFormat.INTERLEAVED)
                plsc.addupdate_scatter(
                    agg_v, [jnp.zeros((L,), jnp.int32), dstv], m0)
                plsc.addupdate_scatter(
                    agg_v, [jnp.full((L,), 1, jnp.int32), dstv], m1)

        def step(it, carry):
            @plsc.parallel_loop(0, N // L, unroll=8)
            def zb(i):
                z16 = jnp.zeros((L,), jnp.float32)
                for r in range(FPT):
                    agg_v[r, pl.ds(i * L, L)] = z16

            start(0, 0)

            def pair(t, cc):
                chA = 2 * t
                start(chA + 1, 1)
                drain(0)
                compute(0)
                start(lax.rem(chA + 2, PNCH), 0)
                drain(1)
                compute(1)
                return cc
            lax.fori_loop(0, PNCH // 2, pair, None)
            drain(0)

            @plsc.parallel_loop(0, N // L, unroll=8)
            def ub(i):
                o0 = (BETA * agg_v[0, pl.ds(i * L, L)] +
                      (1.0 - BETA) * z_v[0, pl.ds(i * L, L)])
                o1 = (BETA * agg_v[1, pl.ds(i * L, L)] +
                      (1.0 - BETA) * z_v[1, pl.ds(i * L, L)])
                pk = plsc.pack(o0, o1, format=plsc.PackFormat.INTERLEAVED)
                out_p[pl.ds(i * L, L)] = plsc.bitcast(pk, jnp.int32)
            return carry

        lax.fori_loop(0, K, step, None)

        @plsc.parallel_loop(0, N // L, unroll=8)
        def fin(i):
            for r in range(FPT):
                agg_v[r, pl.ds(i * L, L)] = (
                    BETA * agg_v[r, pl.ds(i * L, L)] +
                    (1.0 - BETA) * z_v[r, pl.ds(i * L, L)])
        pltpu.sync_copy(agg_v, outT_hbm.at[pl.ds(FPT * wid, FPT)])


def _post_body(ot_ref, logp_ref, ent_ref):
    o = ot_ref[...].T
    m = jnp.max(o, axis=1, keepdims=True)
    ex = jnp.exp(o - m)
    se = jnp.sum(ex, axis=1, keepdims=True)
    logp = o - m - jnp.log(se)
    logp_ref[...] = logp
    q = ex / se
    ent = -jnp.sum(q * jnp.log(q + 1e-16)) / N
    ent_ref[...] = jnp.full((1, 1), ent, jnp.float32)


def _post(outT):
    return pl.pallas_call(
        _post_body,
        out_shape=[
            jax.ShapeDtypeStruct((N, C), jnp.float32),
            jax.ShapeDtypeStruct((1, 1), jnp.float32),
        ],
    )(outT)


def kernel(x, edge_index, train_mask, W1, b1, W2, b2, is_debug):
    zT, p_pack = _mlp(x, W1, b1, W2, b2)
    src = edge_index[0]
    dst = edge_index[1]
    att, dpart = _sc_attention_build()(_sc_attention)(
        p_pack, src.reshape(E // ASUB, ASUB), dst.reshape(E // ASUB, ASUB))
    alpha = _sc_alpha_build()(_sc_alpha)(dpart, dst, att)
    sd = jnp.left_shift(src, 14) | dst
    outT = _sc_diffuse_build()(_sc_diffuse)(zT, sd, alpha)
    logp, ent11 = _post(outT)
    return (logp, ent11[0, 0], att)
